# baseline (speedup 1.0000x reference)
import functools

import jax
import jax.numpy as jnp
from jax import lax
from jax.experimental import pallas as pl
from jax.experimental.pallas import tpu as pltpu
from jax.experimental.pallas import tpu_sc as plsc

NU = 10000
NN = 2 * NU
E = 320000
H = 128
FIN = 64
L = 3

NCORES, NSUB = 2, 16
NW = NCORES * NSUB
DCH = 100
NCHD = (2 * E) // NW // DCH
ACH = 50
NCHA = E // NW // ACH
NBUF = 4
PD = NBUF - 1
assert NW * NCHD * DCH == 2 * E and NW * NCHA * ACH == E
SPU = 632
SPN = 1256
PADU = NSUB * SPU
PADN = NSUB * SPN
IBLK = 40
ZCH = 24
ZN = SPU // ZCH
ZREM = SPU - ZN * ZCH


def _deg_body(dst_hbm, ones_hbm, zeros_hbm, deg_out, idx_v, ones_v, strip_v,
              accum, sem):
    c = lax.axis_index("c")
    s = lax.axis_index("s")
    wid = s * NCORES + c
    stripe = pl.ds(s * SPN, SPN)
    pltpu.sync_copy(zeros_hbm, strip_v)
    pltpu.sync_copy(strip_v, accum.at[stripe])
    pltpu.sync_copy(ones_hbm, ones_v)
    pltpu.sync_copy(dst_hbm.at[wid], idx_v)
    plsc.subcore_barrier()

    @pl.loop(0, NCHD // 8)
    def _(b):
        ds = [pltpu.async_copy(ones_v, accum.at[idx_v.at[b * 8 + k]],
                               sem, add=True) for k in range(8)]
        for d in ds:
            d.wait()

    plsc.subcore_barrier()
    pltpu.sync_copy(accum.at[stripe], strip_v)
    pltpu.sync_copy(strip_v, deg_out.at[c, stripe])


def _agg_body(src_hbm, dst_hbm, table_hbm, zeros_hbm, out_hbm,
              idxs_v, idxd_v, rows_a, rows_b, rows_c, rows_d,
              zbuf_a, zbuf_b, accum,
              sem_ga, sem_gb, sem_gc, sem_gd,
              sem_sa, sem_sb, sem_sc, sem_sd):
    c = lax.axis_index("c")
    s = lax.axis_index("s")
    wid = s * NCORES + c
    sbase = s * SPU

    pltpu.sync_copy(zeros_hbm, zbuf_a)
    zs = [pltpu.async_copy(zbuf_a, accum.at[pl.ds(sbase + k * ZCH, ZCH)],
                           sem_sa) for k in range(ZN)]
    zs.append(pltpu.async_copy(zbuf_a.at[pl.ds(0, ZREM)],
                               accum.at[pl.ds(sbase + ZN * ZCH, ZREM)],
                               sem_sa))
    for d in zs:
        d.wait()
    plsc.subcore_barrier()

    bufs = [(rows_a, sem_ga, sem_sa), (rows_b, sem_gb, sem_sb),
            (rows_c, sem_gc, sem_sc), (rows_d, sem_gd, sem_sd)]

    @pl.loop(0, NCHA // IBLK)
    def _(b):
        pltpu.sync_copy(src_hbm.at[wid, pl.ds(b * IBLK, IBLK)], idxs_v)
        pltpu.sync_copy(dst_hbm.at[wid, pl.ds(b * IBLK, IBLK)], idxd_v)
        g = {}
        sc = {i: None for i in range(NBUF)}
        for i in range(PD):
            g[i] = pltpu.async_copy(table_hbm.at[idxs_v.at[i]],
                                    bufs[i][0], bufs[i][1])
        for t in range(IBLK):
            i = t % NBUF
            g[i].wait()
            sc[i] = pltpu.async_copy(bufs[i][0], accum.at[idxd_v.at[t]],
                                     bufs[i][2], add=True)
            if t + PD < IBLK:
                ip = (t + PD) % NBUF
                if sc[ip] is not None:
                    sc[ip].wait()
                    sc[ip] = None
                g[ip] = pltpu.async_copy(table_hbm.at[idxs_v.at[t + PD]],
                                         bufs[ip][0], bufs[ip][1])
        for i in range(NBUF):
            if sc[i] is not None:
                sc[i].wait()

    plsc.subcore_barrier()

    def rd(k, buf, sem):
        n = ZCH if k < ZN else ZREM
        return pltpu.async_copy(accum.at[pl.ds(sbase + k * ZCH, n)],
                                buf.at[pl.ds(0, n)], sem)

    def wr(k, buf, sem):
        n = ZCH if k < ZN else ZREM
        return pltpu.async_copy(buf.at[pl.ds(0, n)],
                                out_hbm.at[c, pl.ds(sbase + k * ZCH, n)], sem)

    ra = rd(0, zbuf_a, sem_ga)
    rb = rd(1, zbuf_b, sem_gb)
    wa = wb = None
    for k in range(0, ZN + 1, 2):
        ra.wait()
        wa = wr(k, zbuf_a, sem_sa)
        if k + 1 <= ZN:
            rb.wait()
            wb = wr(k + 1, zbuf_b, sem_sb)
        if k + 2 <= ZN:
            wa.wait()
            ra = rd(k + 2, zbuf_a, sem_ga)
            if k + 3 <= ZN:
                wb.wait()
                rb = rd(k + 3, zbuf_b, sem_gb)
    wa.wait()
    if wb is not None:
        wb.wait()


@functools.cache
def _sc_kernels():
    mesh = plsc.VectorSubcoreMesh(core_axis_name="c", subcore_axis_name="s",
                                  num_cores=NCORES, num_subcores=NSUB)
    params = pltpu.CompilerParams(use_tc_tiling_on_sc=False)
    deg = pl.kernel(
        _deg_body,
        compiler_params=params,
        out_type=jax.ShapeDtypeStruct((NCORES, PADN, 16), jnp.float32),
        mesh=mesh,
        scratch_types=[
            pltpu.VMEM((NCHD, DCH), jnp.int32),
            pltpu.VMEM((DCH, 16), jnp.float32),
            pltpu.VMEM((SPN, 16), jnp.float32),
            pltpu.VMEM_SHARED((PADN, 16), jnp.float32),
            pltpu.SemaphoreType.DMA,
        ],
    )
    agg = pl.kernel(
        _agg_body,
        compiler_params=params,
        out_type=jax.ShapeDtypeStruct((NCORES, PADU, H), jnp.float32),
        mesh=mesh,
        scratch_types=[
            pltpu.VMEM((IBLK, ACH), jnp.int32),
            pltpu.VMEM((IBLK, ACH), jnp.int32),
            pltpu.VMEM((ACH, H), jnp.float32),
            pltpu.VMEM((ACH, H), jnp.float32),
            pltpu.VMEM((ACH, H), jnp.float32),
            pltpu.VMEM((ACH, H), jnp.float32),
            pltpu.VMEM((ZCH, H), jnp.float32),
            pltpu.VMEM((ZCH, H), jnp.float32),
            pltpu.VMEM_SHARED((PADU, H), jnp.float32),
            pltpu.SemaphoreType.DMA,
            pltpu.SemaphoreType.DMA,
            pltpu.SemaphoreType.DMA,
            pltpu.SemaphoreType.DMA,
            pltpu.SemaphoreType.DMA,
            pltpu.SemaphoreType.DMA,
            pltpu.SemaphoreType.DMA,
            pltpu.SemaphoreType.DMA,
        ],
    )
    return deg, agg


def _deg_call(dst_all, ones16, zeros16):
    return _sc_kernels()[0](dst_all, ones16, zeros16)


def _agg_call(src, dst, table, zerosH):
    return _sc_kernels()[1](src, dst, table, zerosH)


_BS = 1000
_NB = NU // _BS


def _dinv_from(deg_ref):
    deg = deg_ref[0, :, 0:1] + deg_ref[1, :, 0:1]
    return lax.rsqrt(deg)


def _proj_body(x_ref, wp_ref, bp_ref, deg_ref, w0_ref, x0_ref, hp0_ref):
    x = x_ref[...]
    xp = lax.dot_general(x, wp_ref[...], (((1,), (1,)), ((), ())),
                         preferred_element_type=jnp.float32) + bp_ref[0]
    x0_ref[...] = xp
    h = lax.dot_general(xp, w0_ref[...], (((1,), (1,)), ((), ())),
                        preferred_element_type=jnp.float32)
    hp0_ref[...] = h * _dinv_from(deg_ref)


def _proj_call(x, wp, bp, deg_part, w0, rel):
    roff = rel * _NB
    return pl.pallas_call(
        _proj_body,
        grid=(_NB,),
        in_specs=[
            pl.BlockSpec((_BS, FIN), lambda i: (i, 0)),
            pl.BlockSpec((H, FIN), lambda i: (0, 0)),
            pl.BlockSpec((1, H), lambda i: (0, 0)),
            pl.BlockSpec((NCORES, _BS, 16), lambda i: (0, roff + i, 0)),
            pl.BlockSpec((H, H), lambda i: (0, 0)),
        ],
        out_specs=[
            pl.BlockSpec((_BS, H), lambda i: (i, 0)),
            pl.BlockSpec((_BS, H), lambda i: (i, 0)),
        ],
        out_shape=[jax.ShapeDtypeStruct((NU, H), jnp.float32)] * 2,
    )(x, wp, bp.reshape(1, H), deg_part, w0)


def _post_compute(x_ref, agg_ref, hp_ref, deg_ref, cb_ref, lg_ref, lb_ref):
    dinv = _dinv_from(deg_ref)
    o = (agg_ref[0] + agg_ref[1] + hp_ref[...]) * dinv + cb_ref[0]
    mu = jnp.mean(o, axis=-1, keepdims=True)
    var = jnp.mean((o - mu) ** 2, axis=-1, keepdims=True)
    o = (o - mu) * lax.rsqrt(var + 1e-5) * lg_ref[0] + lb_ref[0]
    return x_ref[...] + jnp.maximum(o, 0.0), dinv


def _post_body(x_ref, agg_ref, hp_ref, deg_ref, cb_ref, lg_ref, lb_ref,
               wn_ref, xn_ref, hpn_ref):
    xn, dinv = _post_compute(x_ref, agg_ref, hp_ref, deg_ref, cb_ref, lg_ref,
                             lb_ref)
    xn_ref[...] = xn
    h = lax.dot_general(xn, wn_ref[...], (((1,), (1,)), ((), ())),
                        preferred_element_type=jnp.float32)
    hpn_ref[...] = h * dinv


def _post_body_last(x_ref, agg_ref, hp_ref, deg_ref, cb_ref, lg_ref, lb_ref,
                    wn_ref, xn_ref):
    xn, _ = _post_compute(x_ref, agg_ref, hp_ref, deg_ref, cb_ref, lg_ref,
                          lb_ref)
    xn_ref[...] = xn


def _post_call(x_cur, agg, hp, deg_part, cb, lg, lb, wn, rel, last):
    roff = rel * _NB
    row_spec = pl.BlockSpec((_BS, H), lambda i: (i, 0))
    vec_spec = pl.BlockSpec((1, H), lambda i: (0, 0))
    out_shape = [jax.ShapeDtypeStruct((NU, H), jnp.float32)]
    out_specs = [row_spec]
    if not last:
        out_shape = out_shape * 2
        out_specs = out_specs * 2
    return pl.pallas_call(
        _post_body_last if last else _post_body,
        grid=(_NB,),
        in_specs=[
            row_spec,
            pl.BlockSpec((NCORES, _BS, H), lambda i: (0, i, 0)),
            row_spec,
            pl.BlockSpec((NCORES, _BS, 16), lambda i: (0, roff + i, 0)),
            vec_spec, vec_spec, vec_spec,
            pl.BlockSpec((H, H), lambda i: (0, 0)),
        ],
        out_specs=out_specs,
        out_shape=out_shape,
    )(x_cur, agg, hp, deg_part, cb.reshape(1, H), lg.reshape(1, H),
      lb.reshape(1, H), wn)


def _proj_small_body(x_ref, w_ref, b_ref, o_ref):
    o_ref[...] = lax.dot_general(x_ref[...], w_ref[...], (((1,), (1,)), ((), ())),
                                 preferred_element_type=jnp.float32) + b_ref[0]


def _proj_small(x, w, b):
    n = x.shape[0]
    return pl.pallas_call(
        _proj_small_body,
        out_shape=jax.ShapeDtypeStruct((n, H), jnp.float32),
    )(x, w, b.reshape(1, H))


def kernel(x_user, x_video, x_category, x_parent_category,
           Wp_user, bp_user, Wp_video, bp_video,
           Wp_category, bp_category, Wp_parent, bp_parent,
           conv_W, conv_b, ln_g, ln_b,
           edge_index_user, edge_index_video):
    su, du = edge_index_user[0], edge_index_user[1]
    sv, dv = edge_index_video[0], edge_index_video[1]

    dst_all = jnp.concatenate([du, dv + NU]).reshape(NW, NCHD, DCH)
    src_e = (su.reshape(NW, NCHA, ACH), sv.reshape(NW, NCHA, ACH))
    dst_e = (du.reshape(NW, NCHA, ACH), dv.reshape(NW, NCHA, ACH))
    ones16 = jnp.ones((DCH, 16), jnp.float32)
    zeros16 = jnp.zeros((SPN, 16), jnp.float32)
    zerosH = jnp.zeros((ZCH, H), jnp.float32)

    deg_part = _deg_call(dst_all, ones16, zeros16)

    x = [None, None]
    hp = [None, None]
    x[0], hp[0] = _proj_call(x_user, Wp_user, bp_user, deg_part, conv_W[0], 0)
    x[1], hp[1] = _proj_call(x_video, Wp_video, bp_video, deg_part,
                             conv_W[0], 1)

    for i in range(L):
        last = i == L - 1
        wn = conv_W[i + 1] if not last else conv_W[0]
        agg = [None, None]
        for r in range(2):
            agg[r] = _agg_call(src_e[r], dst_e[r], hp[r], zerosH)
        for r in range(2):
            res = _post_call(x[r], agg[r], hp[r], deg_part, conv_b[i],
                             ln_g[i], ln_b[i], wn, r, last)
            if last:
                x[r] = res[0]
            else:
                x[r], hp[r] = res

    xc = _proj_small(x_category, Wp_category, bp_category)
    xp = _proj_small(x_parent_category, Wp_parent, bp_parent)
    return (x[0], x[1], xc, xp)


# speedup vs baseline: 1.0416x; 1.0416x over previous
import functools

import jax
import jax.numpy as jnp
from jax import lax
from jax.experimental import pallas as pl
from jax.experimental.pallas import tpu as pltpu
from jax.experimental.pallas import tpu_sc as plsc

NU = 10000
NN = 2 * NU
E = 320000
H = 128
FIN = 64
L = 3

NCORES, NSUB = 2, 16
NW = NCORES * NSUB
DCH = 100
NCHD = (2 * E) // NW // DCH
ACH = 80
NCHA = E // NW // ACH
assert NW * NCHD * DCH == 2 * E and NW * NCHA * ACH == E
SPU = 632
SPN = 1256
PADU = NSUB * SPU
PADN = NSUB * SPN
IBLK = 25
ZCH = 24
ZN = SPU // ZCH
ZREM = SPU - ZN * ZCH


def _deg_body(dst_hbm, ones_hbm, zeros_hbm, deg_out, idx_v, ones_v, strip_v,
              accum, sem):
    c = lax.axis_index("c")
    s = lax.axis_index("s")
    wid = s * NCORES + c
    stripe = pl.ds(s * SPN, SPN)
    pltpu.sync_copy(zeros_hbm, strip_v)
    pltpu.sync_copy(strip_v, accum.at[stripe])
    pltpu.sync_copy(ones_hbm, ones_v)
    pltpu.sync_copy(dst_hbm.at[wid], idx_v)
    plsc.subcore_barrier()

    @pl.loop(0, NCHD // 8)
    def _(b):
        ds = [pltpu.async_copy(ones_v, accum.at[idx_v.at[b * 8 + k]],
                               sem, add=True) for k in range(8)]
        for d in ds:
            d.wait()

    plsc.subcore_barrier()
    pltpu.sync_copy(accum.at[stripe], strip_v)
    pltpu.sync_copy(strip_v, deg_out.at[c, stripe])


def _agg_body(src_hbm, dst_hbm, table_hbm, zeros_hbm, out_hbm,
              idxs_v, idxd_v, rows_a, rows_b, rows_c, zbuf_a, zbuf_b, accum,
              sem_ga, sem_gb, sem_gc, sem_sa, sem_sb, sem_sc):
    c = lax.axis_index("c")
    s = lax.axis_index("s")
    wid = s * NCORES + c
    sbase = s * SPU

    pltpu.sync_copy(zeros_hbm, zbuf_a)
    zs = [pltpu.async_copy(zbuf_a, accum.at[pl.ds(sbase + k * ZCH, ZCH)],
                           sem_sa) for k in range(ZN)]
    zs.append(pltpu.async_copy(zbuf_a.at[pl.ds(0, ZREM)],
                               accum.at[pl.ds(sbase + ZN * ZCH, ZREM)],
                               sem_sa))
    for d in zs:
        d.wait()
    plsc.subcore_barrier()

    bufs = [(rows_a, sem_ga, sem_sa), (rows_b, sem_gb, sem_sb),
            (rows_c, sem_gc, sem_sc)]

    @pl.loop(0, NCHA // IBLK)
    def _(b):
        pltpu.sync_copy(src_hbm.at[wid, pl.ds(b * IBLK, IBLK)], idxs_v)
        pltpu.sync_copy(dst_hbm.at[wid, pl.ds(b * IBLK, IBLK)], idxd_v)
        g = {}
        sc = {0: None, 1: None, 2: None}
        for i in range(2):
            g[i] = pltpu.async_copy(table_hbm.at[idxs_v.at[i]],
                                    bufs[i][0], bufs[i][1])
        for t in range(IBLK):
            i = t % 3
            g[i].wait()
            sc[i] = pltpu.async_copy(bufs[i][0], accum.at[idxd_v.at[t]],
                                     bufs[i][2], add=True)
            if t + 2 < IBLK:
                ip = (t + 2) % 3
                if sc[ip] is not None:
                    sc[ip].wait()
                    sc[ip] = None
                g[ip] = pltpu.async_copy(table_hbm.at[idxs_v.at[t + 2]],
                                         bufs[ip][0], bufs[ip][1])
        for i in range(3):
            if sc[i] is not None:
                sc[i].wait()

    plsc.subcore_barrier()

    def rd(k, buf, sem):
        n = ZCH if k < ZN else ZREM
        return pltpu.async_copy(accum.at[pl.ds(sbase + k * ZCH, n)],
                                buf.at[pl.ds(0, n)], sem)

    def wr(k, buf, sem):
        n = ZCH if k < ZN else ZREM
        return pltpu.async_copy(buf.at[pl.ds(0, n)],
                                out_hbm.at[c, pl.ds(sbase + k * ZCH, n)], sem)

    ra = rd(0, zbuf_a, sem_ga)
    rb = rd(1, zbuf_b, sem_gb)
    wa = wb = None
    for k in range(0, ZN + 1, 2):
        ra.wait()
        wa = wr(k, zbuf_a, sem_sa)
        if k + 1 <= ZN:
            rb.wait()
            wb = wr(k + 1, zbuf_b, sem_sb)
        if k + 2 <= ZN:
            wa.wait()
            ra = rd(k + 2, zbuf_a, sem_ga)
            if k + 3 <= ZN:
                wb.wait()
                rb = rd(k + 3, zbuf_b, sem_gb)
    wa.wait()
    if wb is not None:
        wb.wait()


@functools.cache
def _sc_kernels():
    mesh = plsc.VectorSubcoreMesh(core_axis_name="c", subcore_axis_name="s",
                                  num_cores=NCORES, num_subcores=NSUB)
    params = pltpu.CompilerParams(use_tc_tiling_on_sc=False)
    deg = pl.kernel(
        _deg_body,
        compiler_params=params,
        out_type=jax.ShapeDtypeStruct((NCORES, PADN, 16), jnp.float32),
        mesh=mesh,
        scratch_types=[
            pltpu.VMEM((NCHD, DCH), jnp.int32),
            pltpu.VMEM((DCH, 16), jnp.float32),
            pltpu.VMEM((SPN, 16), jnp.float32),
            pltpu.VMEM_SHARED((PADN, 16), jnp.float32),
            pltpu.SemaphoreType.DMA,
        ],
    )
    agg = pl.kernel(
        _agg_body,
        compiler_params=params,
        out_type=jax.ShapeDtypeStruct((NCORES, PADU, H), jnp.float32),
        mesh=mesh,
        scratch_types=[
            pltpu.VMEM((IBLK, ACH), jnp.int32),
            pltpu.VMEM((IBLK, ACH), jnp.int32),
            pltpu.VMEM((ACH, H), jnp.float32),
            pltpu.VMEM((ACH, H), jnp.float32),
            pltpu.VMEM((ACH, H), jnp.float32),
            pltpu.VMEM((ZCH, H), jnp.float32),
            pltpu.VMEM((ZCH, H), jnp.float32),
            pltpu.VMEM_SHARED((PADU, H), jnp.float32),
            pltpu.SemaphoreType.DMA,
            pltpu.SemaphoreType.DMA,
            pltpu.SemaphoreType.DMA,
            pltpu.SemaphoreType.DMA,
            pltpu.SemaphoreType.DMA,
            pltpu.SemaphoreType.DMA,
        ],
    )
    return deg, agg


def _deg_call(dst_all, ones16, zeros16):
    return _sc_kernels()[0](dst_all, ones16, zeros16)


def _agg_call(src, dst, table, zerosH):
    return _sc_kernels()[1](src, dst, table, zerosH)


_BS = 1000
_NB = NU // _BS


def _dinv_from(deg_ref):
    deg = deg_ref[0, :, 0:1] + deg_ref[1, :, 0:1]
    return lax.rsqrt(deg)


def _proj_body(x_ref, wp_ref, bp_ref, deg_ref, w0_ref, x0_ref, hp0_ref):
    x = x_ref[...]
    xp = lax.dot_general(x, wp_ref[...], (((1,), (1,)), ((), ())),
                         preferred_element_type=jnp.float32) + bp_ref[0]
    x0_ref[...] = xp
    h = lax.dot_general(xp, w0_ref[...], (((1,), (1,)), ((), ())),
                        preferred_element_type=jnp.float32)
    hp0_ref[...] = h * _dinv_from(deg_ref)


def _proj_call(x, wp, bp, deg_part, w0, rel):
    roff = rel * _NB
    return pl.pallas_call(
        _proj_body,
        grid=(_NB,),
        in_specs=[
            pl.BlockSpec((_BS, FIN), lambda i: (i, 0)),
            pl.BlockSpec((H, FIN), lambda i: (0, 0)),
            pl.BlockSpec((1, H), lambda i: (0, 0)),
            pl.BlockSpec((NCORES, _BS, 16), lambda i: (0, roff + i, 0)),
            pl.BlockSpec((H, H), lambda i: (0, 0)),
        ],
        out_specs=[
            pl.BlockSpec((_BS, H), lambda i: (i, 0)),
            pl.BlockSpec((_BS, H), lambda i: (i, 0)),
        ],
        out_shape=[jax.ShapeDtypeStruct((NU, H), jnp.float32)] * 2,
    )(x, wp, bp.reshape(1, H), deg_part, w0)


def _post_compute(x_ref, agg_ref, hp_ref, deg_ref, cb_ref, lg_ref, lb_ref):
    dinv = _dinv_from(deg_ref)
    o = (agg_ref[0] + agg_ref[1] + hp_ref[...]) * dinv + cb_ref[0]
    mu = jnp.mean(o, axis=-1, keepdims=True)
    var = jnp.mean((o - mu) ** 2, axis=-1, keepdims=True)
    o = (o - mu) * lax.rsqrt(var + 1e-5) * lg_ref[0] + lb_ref[0]
    return x_ref[...] + jnp.maximum(o, 0.0), dinv


def _post_body(x_ref, agg_ref, hp_ref, deg_ref, cb_ref, lg_ref, lb_ref,
               wn_ref, xn_ref, hpn_ref):
    xn, dinv = _post_compute(x_ref, agg_ref, hp_ref, deg_ref, cb_ref, lg_ref,
                             lb_ref)
    xn_ref[...] = xn
    h = lax.dot_general(xn, wn_ref[...], (((1,), (1,)), ((), ())),
                        preferred_element_type=jnp.float32)
    hpn_ref[...] = h * dinv


def _post_body_last(x_ref, agg_ref, hp_ref, deg_ref, cb_ref, lg_ref, lb_ref,
                    wn_ref, xn_ref):
    xn, _ = _post_compute(x_ref, agg_ref, hp_ref, deg_ref, cb_ref, lg_ref,
                          lb_ref)
    xn_ref[...] = xn


def _post_call(x_cur, agg, hp, deg_part, cb, lg, lb, wn, rel, last):
    roff = rel * _NB
    row_spec = pl.BlockSpec((_BS, H), lambda i: (i, 0))
    vec_spec = pl.BlockSpec((1, H), lambda i: (0, 0))
    out_shape = [jax.ShapeDtypeStruct((NU, H), jnp.float32)]
    out_specs = [row_spec]
    if not last:
        out_shape = out_shape * 2
        out_specs = out_specs * 2
    return pl.pallas_call(
        _post_body_last if last else _post_body,
        grid=(_NB,),
        in_specs=[
            row_spec,
            pl.BlockSpec((NCORES, _BS, H), lambda i: (0, i, 0)),
            row_spec,
            pl.BlockSpec((NCORES, _BS, 16), lambda i: (0, roff + i, 0)),
            vec_spec, vec_spec, vec_spec,
            pl.BlockSpec((H, H), lambda i: (0, 0)),
        ],
        out_specs=out_specs,
        out_shape=out_shape,
    )(x_cur, agg, hp, deg_part, cb.reshape(1, H), lg.reshape(1, H),
      lb.reshape(1, H), wn)


def _proj_small_body(x_ref, w_ref, b_ref, o_ref):
    o_ref[...] = lax.dot_general(x_ref[...], w_ref[...], (((1,), (1,)), ((), ())),
                                 preferred_element_type=jnp.float32) + b_ref[0]


def _proj_small(x, w, b):
    n = x.shape[0]
    return pl.pallas_call(
        _proj_small_body,
        out_shape=jax.ShapeDtypeStruct((n, H), jnp.float32),
    )(x, w, b.reshape(1, H))


def kernel(x_user, x_video, x_category, x_parent_category,
           Wp_user, bp_user, Wp_video, bp_video,
           Wp_category, bp_category, Wp_parent, bp_parent,
           conv_W, conv_b, ln_g, ln_b,
           edge_index_user, edge_index_video):
    su, du = edge_index_user[0], edge_index_user[1]
    sv, dv = edge_index_video[0], edge_index_video[1]

    dst_all = jnp.concatenate([du, dv + NU]).reshape(NW, NCHD, DCH)
    src_e = (su.reshape(NW, NCHA, ACH), sv.reshape(NW, NCHA, ACH))
    dst_e = (du.reshape(NW, NCHA, ACH), dv.reshape(NW, NCHA, ACH))
    ones16 = jnp.ones((DCH, 16), jnp.float32)
    zeros16 = jnp.zeros((SPN, 16), jnp.float32)
    zerosH = jnp.zeros((ZCH, H), jnp.float32)

    deg_part = _deg_call(dst_all, ones16, zeros16)

    x = [None, None]
    hp = [None, None]
    x[0], hp[0] = _proj_call(x_user, Wp_user, bp_user, deg_part, conv_W[0], 0)
    x[1], hp[1] = _proj_call(x_video, Wp_video, bp_video, deg_part,
                             conv_W[0], 1)

    for i in range(L):
        last = i == L - 1
        wn = conv_W[i + 1] if not last else conv_W[0]
        agg = [None, None]
        for r in range(2):
            agg[r] = _agg_call(src_e[r], dst_e[r], hp[r], zerosH)
        for r in range(2):
            res = _post_call(x[r], agg[r], hp[r], deg_part, conv_b[i],
                             ln_g[i], ln_b[i], wn, r, last)
            if last:
                x[r] = res[0]
            else:
                x[r], hp[r] = res

    xc = _proj_small(x_category, Wp_category, bp_category)
    xp = _proj_small(x_parent_category, Wp_parent, bp_parent)
    return (x[0], x[1], xc, xp)
